# Initial kernel scaffold; baseline (speedup 1.0000x reference)
import jax
import jax.numpy as jnp
from jax.experimental import pallas as pl


def kernel(x, r_ij, neighbors, neighbor_mask, f_ij, W1, b1, W2, b2, Win, Wout, bout, Wd, bd):
    raise NotImplementedError("write your pallas kernel here")


# speedup vs baseline: 12.7490x; 12.7490x over previous
import functools
import jax
import jax.numpy as jnp
from jax import lax
from jax.experimental import pallas as pl
from jax.experimental.pallas import tpu as pltpu

_LN2 = 0.6931471805599453
_CUTOFF = 5.0


def _ssp(v):
    return jax.nn.softplus(v) - _LN2


def _body(x_ref, r_ref, nb_ref, mask_ref, f_ref,
          W1_ref, b1_ref, W2_ref, b2_ref, Win_ref,
          Wout_ref, bout_ref, Wd_ref, bd_ref,
          out_ref, y_s, *, N, Tn, K):
    nt = pl.program_id(1)

    @pl.when(nt == 0)
    def _():
        y_s[...] = jnp.dot(x_ref[0], Win_ref[...],
                           preferred_element_type=jnp.float32)

    E = Tn * K
    f = f_ref[0]
    h = _ssp(jnp.dot(f, W1_ref[...], preferred_element_type=jnp.float32)
             + b1_ref[...])
    wf = jnp.dot(h, W2_ref[...], preferred_element_type=jnp.float32) \
        + b2_ref[...]
    m = jnp.where(r_ref[0, 0, 0] <= _CUTOFF, 1.0, 0.0) * mask_ref[0, 0, 0]
    wf = wf * m[:, None]

    idx = nb_ref[0, 0, 0]
    onehot = (idx[:, None] ==
              lax.broadcasted_iota(jnp.int32, (E, N), 1)).astype(jnp.float32)
    yg = jnp.dot(onehot, y_s[...], preferred_element_type=jnp.float32)

    acc = jnp.sum((yg * wf).reshape(Tn, K, -1), axis=1)
    z = _ssp(jnp.dot(acc, Wout_ref[...], preferred_element_type=jnp.float32)
             + bout_ref[...])
    out_ref[0] = jnp.dot(z, Wd_ref[...],
                         preferred_element_type=jnp.float32) + bd_ref[...]


@functools.partial(jax.jit, static_argnames=("interpret",))
def kernel(x, r_ij, neighbors, neighbor_mask, f_ij,
           W1, b1, W2, b2, Win, Wout, bout, Wd, bd, interpret=False):
    B, N, K = neighbors.shape
    Din = x.shape[-1]
    Fs = f_ij.shape[-1]
    Ff = W2.shape[-1]
    Dout = Wd.shape[-1]

    Tn = 64
    NT = N // Tn
    E = Tn * K

    f2 = f_ij.reshape(B, N * K, Fs)
    r2 = r_ij.reshape(B, NT, 1, E)
    nb2 = neighbors.astype(jnp.int32).reshape(B, NT, 1, E)
    mk2 = neighbor_mask.reshape(B, NT, 1, E)

    full = lambda s: pl.BlockSpec(s, lambda b, nt: (0,) * len(s))
    grid = (B, NT)

    out = pl.pallas_call(
        functools.partial(_body, N=N, Tn=Tn, K=K),
        grid=grid,
        in_specs=[
            pl.BlockSpec((1, N, Din), lambda b, nt: (b, 0, 0)),
            pl.BlockSpec((1, 1, 1, E), lambda b, nt: (b, nt, 0, 0)),
            pl.BlockSpec((1, 1, 1, E), lambda b, nt: (b, nt, 0, 0)),
            pl.BlockSpec((1, 1, 1, E), lambda b, nt: (b, nt, 0, 0)),
            pl.BlockSpec((1, E, Fs), lambda b, nt: (b, nt, 0)),
            full((Fs, Ff)), full((1, Ff)),
            full((Ff, Ff)), full((1, Ff)),
            full((Din, Ff)),
            full((Ff, Dout)), full((1, Dout)),
            full((Dout, Dout)), full((1, Dout)),
        ],
        out_specs=pl.BlockSpec((1, Tn, Dout), lambda b, nt: (b, nt, 0)),
        out_shape=jax.ShapeDtypeStruct((B, N, Dout), jnp.float32),
        scratch_shapes=[pltpu.VMEM((N, Ff), jnp.float32)],
        compiler_params=pltpu.CompilerParams(
            dimension_semantics=("arbitrary", "arbitrary")),
        interpret=interpret,
    )(x, r2, nb2, mk2, f2,
      W1, b1.reshape(1, Ff), W2, b2.reshape(1, Ff), Win,
      Wout, bout.reshape(1, Dout), Wd, bd.reshape(1, Dout))
    return out


# speedup vs baseline: 13.0587x; 1.0243x over previous
import functools
import jax
import jax.numpy as jnp
from jax import lax
from jax.experimental import pallas as pl
from jax.experimental.pallas import tpu as pltpu
from jax.experimental.pallas import tpu_sc as plsc

_LN2 = 0.6931471805599453
_CUTOFF = 5.0

_NW = 32
_R = 2
_LANES = 16


def _ssp(v):
    return jax.nn.softplus(v) - _LN2


def _proj_body(x_ref, Win_ref, y_ref):
    y_ref[...] = jnp.dot(x_ref[...], Win_ref[...],
                         preferred_element_type=jnp.float32)


def _filter_body(f_ref, r_ref, mask_ref, W1_ref, b1_ref, W2_ref, b2_ref,
                 wf_ref):
    h = _ssp(jnp.dot(f_ref[...], W1_ref[...],
                     preferred_element_type=jnp.float32) + b1_ref[...])
    wf = jnp.dot(h, W2_ref[...], preferred_element_type=jnp.float32) \
        + b2_ref[...]
    m = jnp.where(r_ref[0, 0] <= _CUTOFF, 1.0, 0.0) * mask_ref[0, 0]
    wf_ref[...] = wf * m[:, None]


def _sc_body(y_hbm, wf_hbm, nb_hbm, out_hbm,
             idx_v, yg_v, wf_v, out_v, sem0, sem1, *, N, K, Ff, RPW):
    wid = lax.axis_index("c") * 16 + lax.axis_index("s")
    base_row = wid * RPW
    base_edge = base_row * K
    RK = _R * K
    G = RPW // _R
    NF = Ff // _LANES

    pltpu.sync_copy(nb_hbm.at[wid], idx_v)
    bvec = jnp.full((_LANES,), (base_row // N) * N, jnp.int32)

    @pl.loop(0, (RPW * K) // _LANES)
    def _(j):
        sl = pl.ds(j * _LANES, _LANES)
        idx_v[sl] = idx_v[sl] + bvec

    sems = (sem0, sem1)

    def fire(g, slot):
        pltpu.async_copy(wf_hbm.at[pl.ds(base_edge + g * RK, RK)],
                         wf_v.at[slot], sems[slot])
        pltpu.async_copy(y_hbm.at[idx_v.at[pl.ds(g * RK, RK)]],
                         yg_v.at[slot], sems[slot])

    def drain(slot):
        pltpu.make_async_copy(wf_hbm.at[pl.ds(0, RK)], wf_v.at[slot],
                              sems[slot]).wait()
        pltpu.make_async_copy(wf_hbm.at[pl.ds(0, RK)], yg_v.at[slot],
                              sems[slot]).wait()

    def compute(g, slot):
        for r in range(_R):
            def kstep(k, accs):
                e = r * K + k
                return tuple(
                    accs[fc] + yg_v[slot, e, pl.ds(fc * _LANES, _LANES)]
                    * wf_v[slot, e, pl.ds(fc * _LANES, _LANES)]
                    for fc in range(NF))
            accs = lax.fori_loop(
                0, K, kstep,
                tuple(jnp.zeros((_LANES,), jnp.float32) for _ in range(NF)))
            row = g * _R + r
            for fc in range(NF):
                out_v[row, pl.ds(fc * _LANES, _LANES)] = accs[fc]

    fire(0, 0)

    @pl.loop(0, G, step=2)
    def _(g0):
        for b in range(2):
            g = g0 + b

            @pl.when(g + 1 < G)
            def _():
                fire(g + 1, 1 - b)

            drain(b)
            compute(g, b)

    pltpu.sync_copy(out_v, out_hbm.at[pl.ds(base_row, RPW)])


def _post_body(a_ref, Wout_ref, bout_ref, Wd_ref, bd_ref, o_ref):
    z = _ssp(jnp.dot(a_ref[...], Wout_ref[...],
                     preferred_element_type=jnp.float32) + bout_ref[...])
    o_ref[...] = jnp.dot(z, Wd_ref[...],
                         preferred_element_type=jnp.float32) + bd_ref[...]


@jax.jit
def kernel(x, r_ij, neighbors, neighbor_mask, f_ij,
           W1, b1, W2, b2, Win, Wout, bout, Wd, bd):
    B, N, K = neighbors.shape
    Din = x.shape[-1]
    Fs = f_ij.shape[-1]
    Ff = W2.shape[-1]
    Dout = Wd.shape[-1]
    NE = B * N * K
    RPW = (B * N) // _NW

    y = pl.pallas_call(
        _proj_body,
        in_specs=[pl.BlockSpec((B * N, Din), lambda: (0, 0)),
                  pl.BlockSpec((Din, Ff), lambda: (0, 0))],
        out_specs=pl.BlockSpec((B * N, Ff), lambda: (0, 0)),
        out_shape=jax.ShapeDtypeStruct((B * N, Ff), jnp.float32),
    )(x.reshape(B * N, Din), Win)

    Ew = 4096
    T = NE // Ew
    full = lambda s: pl.BlockSpec(s, lambda t: (0,) * len(s))
    wf = pl.pallas_call(
        _filter_body,
        grid=(T,),
        in_specs=[
            pl.BlockSpec((Ew, Fs), lambda t: (t, 0)),
            pl.BlockSpec((1, 1, Ew), lambda t: (t, 0, 0)),
            pl.BlockSpec((1, 1, Ew), lambda t: (t, 0, 0)),
            full((Fs, Ff)), full((1, Ff)), full((Ff, Ff)), full((1, Ff)),
        ],
        out_specs=pl.BlockSpec((Ew, Ff), lambda t: (t, 0)),
        out_shape=jax.ShapeDtypeStruct((NE, Ff), jnp.float32),
    )(f_ij.reshape(NE, Fs), r_ij.reshape(T, 1, Ew),
      neighbor_mask.reshape(T, 1, Ew),
      W1, b1.reshape(1, Ff), W2, b2.reshape(1, Ff))

    nb = neighbors.astype(jnp.int32).reshape(_NW, RPW * K)
    agg = pl.kernel(
        functools.partial(_sc_body, N=N, K=K, Ff=Ff, RPW=RPW),
        mesh=plsc.VectorSubcoreMesh(core_axis_name="c", subcore_axis_name="s"),
        out_type=jax.ShapeDtypeStruct((B * N, Ff), jnp.float32),
        scratch_types=[
            pltpu.VMEM((RPW * K,), jnp.int32),
            pltpu.VMEM((2, _R * K, Ff), jnp.float32),
            pltpu.VMEM((2, _R * K, Ff), jnp.float32),
            pltpu.VMEM((RPW, Ff), jnp.float32),
            pltpu.SemaphoreType.DMA,
            pltpu.SemaphoreType.DMA,
        ],
    )(y, wf, nb)

    Rw = 2048
    out = pl.pallas_call(
        _post_body,
        grid=((B * N) // Rw,),
        in_specs=[
            pl.BlockSpec((Rw, Ff), lambda t: (t, 0)),
            full((Ff, Dout)), full((1, Dout)),
            full((Dout, Dout)), full((1, Dout)),
        ],
        out_specs=pl.BlockSpec((Rw, Dout), lambda t: (t, 0)),
        out_shape=jax.ShapeDtypeStruct((B * N, Dout), jnp.float32),
    )(agg, Wout, bout.reshape(1, Dout), Wd, bd.reshape(1, Dout))

    return out.reshape(B, N, Dout)
